# baseline (speedup 1.0000x reference)
import functools

import jax
import jax.numpy as jnp
from jax import lax
from jax.experimental import pallas as pl
from jax.experimental.pallas import tpu as pltpu
from jax.experimental.pallas import tpu_sc as plsc

N_SEQ = 50000
N_LAB = 5000
D = 128
E_BT = 100000
E_INC = 100000
E_CT = 400000

NC = 2
NS = 16
L = 16
EW = 2000
DW = 2048

_f32 = jnp.float32
_i32 = jnp.int32

_DEG_JOBS = (
    (E_BT, N_SEQ, 50048),
    (E_BT, N_LAB, 5120),
    (E_INC, N_LAB, 5120),
    (E_INC, N_SEQ, 50048),
    (E_CT, N_SEQ, 50048),
    (E_CT, N_SEQ, 50048),
)


def _deg_body(bt_s, bt_d, inc_s, inc_d, ct_s, ct_d, o0, o1, o2, o3, o4, o5,
              idx2d, ones_v, zv, stg, *accs):
    core = lax.axis_index("c")
    sub = lax.axis_index("s")
    wid = sub * NC + core
    edge_refs = (bt_s, bt_d, inc_s, inc_d, ct_s, ct_d)
    out_refs = (o0, o1, o2, o3, o4, o5)

    def _fill(i, c):
        ones_v[pl.ds(i * L, L)] = jnp.ones((L,), _f32)
        zv[pl.ds(i * L, L)] = jnp.zeros((L,), _f32)
        return c
    lax.fori_loop(0, 128 // L, _fill, 0)

    for (E, N, NP), acc in zip(_DEG_JOBS, accs):
        share = NP // NS
        for j in range(share // 128):
            pltpu.sync_copy(zv, acc.at[pl.ds(sub * share + j * 128, 128)])
        rem = share % 128
        if rem:
            pltpu.sync_copy(zv.at[pl.ds(0, rem)],
                            acc.at[pl.ds(sub * share + share - rem, rem)])
    plsc.subcore_barrier()

    for eref, acc in zip(edge_refs, accs):
        nwin = eref.shape[0]
        nmine = (nwin - wid + 31) // 32

        def _win(i, c, eref=eref, acc=acc):
            w = wid + i * 32
            pltpu.sync_copy(eref.at[w], idx2d)
            for r in range(DW // 128):
                pltpu.sync_copy(ones_v, acc.at[idx2d.at[r]], add=True)
            return c
        lax.fori_loop(0, nmine, _win, 0)
    plsc.subcore_barrier()

    SB = 784
    for (E, N, NP), out, acc in zip(_DEG_JOBS, out_refs, accs):
        share = NP // NS
        for j in range(share // SB):
            pltpu.sync_copy(acc.at[pl.ds(sub * share + j * SB, SB)], stg)
            pltpu.sync_copy(stg,
                            out.at[pl.ds(core * NP + sub * share + j * SB,
                                         SB)])
        rem = share % SB
        if rem:
            off = share - rem
            pltpu.sync_copy(acc.at[pl.ds(sub * share + off, rem)],
                            stg.at[pl.ds(0, rem)])
            pltpu.sync_copy(stg.at[pl.ds(0, rem)],
                            out.at[pl.ds(core * NP + sub * share + off,
                                         rem)])


def _make_deg_kernel():
    out_type = tuple(jax.ShapeDtypeStruct((NC * np_,), _f32)
                     for (_, _, np_) in _DEG_JOBS)
    scratch = [
        pltpu.VMEM((DW // 128, 128), _i32),
        pltpu.VMEM((128,), _f32),
        pltpu.VMEM((128,), _f32),
        pltpu.VMEM((784,), _f32),
    ] + [pltpu.VMEM_SHARED((np_,), _f32) for (_, _, np_) in _DEG_JOBS]
    mesh = plsc.VectorSubcoreMesh(core_axis_name="c", subcore_axis_name="s",
                                  num_cores=NC, num_subcores=NS)
    return pl.kernel(_deg_body, out_type=out_type, mesh=mesh,
                     compiler_params=pltpu.CompilerParams(
                         needs_layout_passes=False),
                     scratch_types=scratch, name="sc_degrees")


def _segsum_stage(feat, src, dst, out, ew_s, ew_d, m_val, srcb, idx2d, rows,
                  zv, gsem, ssem, accum, core, sub, *, E, CHUNK, CPS):
    nwin = E // EW
    nmine = (nwin - sub + NS - 1) // NS
    AROWS = CHUNK + 128
    zshare = AROWS // NS
    fshare = CHUNK // NS
    B = 128
    ZR = zv.shape[0]

    for kk in range(CPS):
        chunk_id = core * CPS + kk
        lo = chunk_id * CHUNK

        for j in range(zshare // ZR):
            pltpu.sync_copy(zv, accum.at[pl.ds(sub * zshare + j * ZR, ZR)])
        rem = zshare % ZR
        if rem:
            pltpu.sync_copy(zv.at[pl.ds(0, rem)],
                            accum.at[pl.ds(sub * zshare + zshare - rem, rem)])
        plsc.subcore_barrier()

        def _win(i, cnt):
            w = sub + i * NS
            pltpu.sync_copy(src.at[pl.ds(w * EW, EW)], ew_s)
            pltpu.sync_copy(dst.at[pl.ds(w * EW, EW)], ew_d)

            def _grp(g, cnt):
                sv = ew_s[pl.ds(g * L, L)]
                dv = ew_d[pl.ds(g * L, L)]
                m = (dv >= lo) & (dv < lo + CHUNK)
                key = jnp.where(m, jnp.int32(0), jnp.int32(1))
                _, v2 = plsc.sort_key_val(key, sv * 16384 + (dv - lo))
                m_val[pl.ds(cnt, L)] = v2
                pc = plsc.all_reduce_population_count(m)
                return cnt + pc[0]
            return lax.fori_loop(0, EW // L, _grp, cnt)
        cnt = lax.fori_loop(0, nmine, _win, jnp.int32(0))

        lanes = lax.iota(_i32, L)
        for k in range(B // L):
            m_val[pl.ds(cnt + k * L, L)] = (
                (lanes + sub * L) * 16384 + (CHUNK + k * L + lanes))
        nb = (cnt + B - 1) // B

        def _build(bi, s):
            for k in range(B // L):
                v = m_val[pl.ds(bi * B + k * L, L)]
                srcb[s, pl.ds(k * L, L)] = lax.shift_right_logical(v, 14)
                idx2d[s, pl.ds(k * L, L)] = jnp.bitwise_and(v, 16383)

        @pl.when(nb >= 1)
        def _():
            _build(jnp.int32(0), jnp.int32(0))
            pltpu.async_copy(feat.at[srcb.at[0]], rows.at[pl.ds(0, B)],
                             gsem.at[0])

        def _batch(j, c):
            s = jnp.bitwise_and(j, 1)
            o = 1 - s
            rs = rows.at[pl.ds(s * B, B)]
            ro = rows.at[pl.ds(o * B, B)]

            @pl.when(j >= 1)
            def _():
                pltpu.make_async_copy(ro, accum.at[idx2d.at[o]], ssem).wait()

            @pl.when(j + 1 < nb)
            def _():
                _build(j + 1, o)
                pltpu.async_copy(feat.at[srcb.at[o]], ro, gsem.at[o])

            pltpu.make_async_copy(feat.at[srcb.at[s]], rs, gsem.at[s]).wait()
            pltpu.async_copy(rs, accum.at[idx2d.at[s]], ssem, add=True)
            return c
        lax.fori_loop(0, nb, _batch, 0)

        @pl.when(nb >= 1)
        def _():
            s = jnp.bitwise_and(nb - 1, 1)
            pltpu.make_async_copy(rows.at[pl.ds(s * B, B)],
                                  accum.at[idx2d.at[s]], ssem).wait()
        plsc.subcore_barrier()

        FB = rows.shape[0]
        for j in range(fshare // FB):
            pltpu.sync_copy(accum.at[pl.ds(sub * fshare + j * FB, FB)], rows)
            pltpu.sync_copy(rows, out.at[pl.ds(lo + sub * fshare + j * FB,
                                               FB)])
        frem = fshare % FB
        if frem:
            foff = fshare - frem
            pltpu.sync_copy(accum.at[pl.ds(sub * fshare + foff, frem)],
                            rows.at[pl.ds(0, frem)])
            pltpu.sync_copy(rows.at[pl.ds(0, frem)],
                            out.at[pl.ds(lo + sub * fshare + foff, frem)])
        plsc.subcore_barrier()


_SEG_JOBS = (
    (E_BT, 2560, 1),
    (E_INC, 6272, 4),
    (E_CT, 6272, 4),
)
PAD_LAB = 2 * 2560
PAD_SEQ = 8 * 6272


def _seg3_body(fA, sA, dA, fB, sB, dB, fC, sC, dC, oA, oB, oC,
               ew_s, ew_d, m_val, srcb, idx2d, rows, zv, gsem, ssem, accum):
    core = lax.axis_index("c")
    sub = lax.axis_index("s")

    def _z(i, c):
        for k in range(D // L):
            zv[i, pl.ds(k * L, L)] = jnp.zeros((L,), _f32)
        return c
    lax.fori_loop(0, zv.shape[0], _z, 0)

    for (E, chunk, cps), feat, src, dst, out in zip(
            _SEG_JOBS, (fA, fB, fC), (sA, sB, sC), (dA, dB, dC),
            (oA, oB, oC)):
        _segsum_stage(feat, src, dst, out, ew_s, ew_d, m_val, srcb, idx2d,
                      rows, zv, gsem, ssem, accum, core, sub,
                      E=E, CHUNK=chunk, CPS=cps)


def _make_seg3_kernel():
    cap = ((E_CT // EW + NS - 1) // NS) * EW + 128
    scratch = [
        pltpu.VMEM((EW,), _i32),
        pltpu.VMEM((EW,), _i32),
        pltpu.VMEM((cap,), _i32),
        pltpu.VMEM((2, 128), _i32),
        pltpu.VMEM((2, 128), _i32),
        pltpu.VMEM((256, D), _f32),
        pltpu.VMEM((32, D), _f32),
        pltpu.SemaphoreType.DMA((2,)),
        pltpu.SemaphoreType.DMA,
        pltpu.VMEM_SHARED((6272 + 128, D), _f32),
    ]
    mesh = plsc.VectorSubcoreMesh(core_axis_name="c", subcore_axis_name="s",
                                  num_cores=NC, num_subcores=NS)
    out_type = (jax.ShapeDtypeStruct((PAD_LAB, D), _f32),
                jax.ShapeDtypeStruct((PAD_SEQ, D), _f32),
                jax.ShapeDtypeStruct((PAD_SEQ, D), _f32))
    return pl.kernel(_seg3_body, out_type=out_type, mesh=mesh,
                     compiler_params=pltpu.CompilerParams(
                         needs_layout_passes=False),
                     scratch_types=scratch, name="sc_segsum3")


def _inv_sqrt(dblk):
    return lax.rsqrt(jnp.maximum(jnp.sum(dblk, axis=1), 1.0))


def _prescale2_body(x_ref, da_ref, db_ref, oa_ref, ob_ref):
    x = x_ref[...]
    oa_ref[...] = x * _inv_sqrt(da_ref[...])[:, None]
    ob_ref[...] = x * _inv_sqrt(db_ref[...])[:, None]


def _prescale1_body(x_ref, da_ref, oa_ref):
    oa_ref[...] = x_ref[...] * _inv_sqrt(da_ref[...])[:, None]


def _prescale(x, degs, blk):
    n = x.shape[0]
    grid = n // blk
    xspec = pl.BlockSpec((blk, D), lambda i: (i, 0))
    dspec = pl.BlockSpec((blk, 2), lambda i: (i, 0))
    if len(degs) == 2:
        return pl.pallas_call(
            _prescale2_body, grid=(grid,),
            in_specs=[xspec, dspec, dspec],
            out_specs=(xspec, xspec),
            out_shape=(jax.ShapeDtypeStruct((n, D), _f32),) * 2,
        )(x, degs[0], degs[1])
    return pl.pallas_call(
        _prescale1_body, grid=(grid,),
        in_specs=[xspec, dspec],
        out_specs=xspec,
        out_shape=jax.ShapeDtypeStruct((n, D), _f32))(x, degs[0])


def _relu_block(a, b, da, db, wa, wb, bias):
    sa = _inv_sqrt(da)[:, None]
    sb = _inv_sqrt(db)[:, None]
    y = (jnp.dot(a * sa, wa, preferred_element_type=_f32)
         + jnp.dot(b * sb, wb, preferred_element_type=_f32)
         + bias[0, :][None, :] + bias[1, :][None, :]) * 0.5
    return jnp.maximum(y, 0.0)


def _post_stats_body(a_ref, b_ref, da_ref, db_ref, wa_ref, wb_ref, bias_ref,
                     stat_ref, acc_ref, *, grid):
    i = pl.program_id(0)
    y = _relu_block(a_ref[...], b_ref[...], da_ref[...], db_ref[...],
                    wa_ref[...], wb_ref[...], bias_ref[...])

    @pl.when(i == 0)
    def _():
        acc_ref[...] = jnp.zeros_like(acc_ref)

    s1 = jnp.sum(y, axis=0)
    s2 = jnp.sum(y * y, axis=0)
    acc_ref[...] += jnp.concatenate([s1[None, :], s2[None, :]], axis=0)

    @pl.when(i == grid - 1)
    def _():
        stat_ref[...] = acc_ref[...]


def _post_apply2_body(a_ref, b_ref, da_ref, db_ref, wa_ref, wb_ref, bias_ref,
                      stat_ref, gb_ref, so1_ref, so2_ref, o1_ref, o2_ref, *,
                      n):
    y = _relu_block(a_ref[...], b_ref[...], da_ref[...], db_ref[...],
                    wa_ref[...], wb_ref[...], bias_ref[...])
    mu = stat_ref[0, :] / n
    var = stat_ref[1, :] / n - mu * mu
    h = (y - mu[None, :]) * lax.rsqrt(var + 1e-5)[None, :]
    h = h * gb_ref[0, :][None, :] + gb_ref[1, :][None, :]
    o1_ref[...] = h * _inv_sqrt(so1_ref[...])[:, None]
    o2_ref[...] = h * _inv_sqrt(so2_ref[...])[:, None]


def _post_apply1_body(a_ref, b_ref, da_ref, db_ref, wa_ref, wb_ref, bias_ref,
                      stat_ref, gb_ref, o1_ref, *, n):
    y = _relu_block(a_ref[...], b_ref[...], da_ref[...], db_ref[...],
                    wa_ref[...], wb_ref[...], bias_ref[...])
    mu = stat_ref[0, :] / n
    var = stat_ref[1, :] / n - mu * mu
    h = (y - mu[None, :]) * lax.rsqrt(var + 1e-5)[None, :]
    o1_ref[...] = h * gb_ref[0, :][None, :] + gb_ref[1, :][None, :]


def _post_block(n, aggs, degs_in, Ws, biases, gamma_beta, out_scale, blk):
    grid = n // blk
    if len(aggs) == 1:
        aggs = (aggs[0], aggs[0])
        degs_in = (degs_in[0], degs_in[0])
        Ws = (Ws[0], Ws[0])
        biases = (biases[0], biases[0])
    bias = jnp.concatenate([biases[0][None, :], biases[1][None, :]], axis=0)
    aspec = pl.BlockSpec((blk, D), lambda i: (i, 0))
    dspec = pl.BlockSpec((blk, 2), lambda i: (i, 0))
    wspec = pl.BlockSpec((D, D), lambda i: (0, 0))
    cspec = pl.BlockSpec((2, D), lambda i: (0, 0))
    args = (aggs[0], aggs[1], degs_in[0], degs_in[1], Ws[0], Ws[1], bias)
    stats = pl.pallas_call(
        functools.partial(_post_stats_body, grid=grid),
        grid=(grid,),
        in_specs=[aspec, aspec, dspec, dspec, wspec, wspec, cspec],
        out_specs=cspec,
        out_shape=jax.ShapeDtypeStruct((2, D), _f32),
        scratch_shapes=[pltpu.VMEM((2, D), _f32)],
    )(*args)
    if out_scale is not None:
        return pl.pallas_call(
            functools.partial(_post_apply2_body, n=float(n)),
            grid=(grid,),
            in_specs=[aspec, aspec, dspec, dspec, wspec, wspec, cspec, cspec,
                      cspec, dspec, dspec],
            out_specs=(aspec, aspec),
            out_shape=(jax.ShapeDtypeStruct((n, D), _f32),) * 2,
        )(*args, stats, gamma_beta, out_scale[0], out_scale[1])
    return pl.pallas_call(
        functools.partial(_post_apply1_body, n=float(n)),
        grid=(grid,),
        in_specs=[aspec, aspec, dspec, dspec, wspec, wspec, cspec, cspec,
                  cspec],
        out_specs=aspec,
        out_shape=jax.ShapeDtypeStruct((n, D), _f32),
    )(*args, stats, gamma_beta)


_deg_kernel = _make_deg_kernel()
_seg3 = _make_seg3_kernel()


def _pad_edges(idx, n_fill):
    e = idx.shape[0]
    ep = (e + DW - 1) // DW * DW
    out = jnp.concatenate([idx, jnp.full((ep - e,), n_fill, _i32)])
    return out.reshape(ep // DW, DW // 128, 128)


def kernel(x_sequence, x_label, bt_src, bt_dst, inc_src, inc_dst, ct_src,
           ct_dst, W_bt1, b_bt1, W_inc1, b_inc1, W_ct1, b_ct1, W_bt2, b_bt2,
           W_inc2, b_inc2, W_ct2, b_ct2, g1s, be1s, g1l, be1l, g2s, be2s,
           g2l, be2l):
    degs = _deg_kernel(_pad_edges(bt_src, N_SEQ), _pad_edges(bt_dst, N_LAB),
                       _pad_edges(inc_src, N_LAB), _pad_edges(inc_dst, N_SEQ),
                       _pad_edges(ct_src, N_SEQ), _pad_edges(ct_dst, N_SEQ))
    dT_bt_s, dT_bt_d, dT_inc_s, dT_inc_d, dT_ct_s, dT_ct_d = (
        d.reshape(NC, -1).T for d in degs)

    feat_bt1, feat_ct1 = _prescale(x_sequence, (dT_bt_s, dT_ct_s), 1000)
    feat_inc1 = _prescale(x_label, (dT_inc_s,), 1000)

    agg_bt1, agg_inc1, agg_ct1 = _seg3(
        feat_bt1, bt_src, bt_dst, feat_inc1, inc_src, inc_dst,
        feat_ct1, ct_src, ct_dst)

    gb1s = jnp.stack([g1s, be1s])
    gb1l = jnp.stack([g1l, be1l])
    feat_bt2, feat_ct2 = _post_block(
        N_SEQ, (agg_inc1, agg_ct1), (dT_inc_d, dT_ct_d), (W_inc1, W_ct1),
        (b_inc1, b_ct1), gb1s, (dT_bt_s, dT_ct_s), 1000)
    feat_inc2 = _post_block(N_LAB, (agg_bt1,), (dT_bt_d,), (W_bt1,),
                            (b_bt1,), gb1l, (dT_inc_s, dT_inc_s), 1000)[0]

    agg_bt2, agg_inc2, agg_ct2 = _seg3(
        feat_bt2, bt_src, bt_dst, feat_inc2, inc_src, inc_dst,
        feat_ct2, ct_src, ct_dst)

    gb2s = jnp.stack([g2s, be2s])
    gb2l = jnp.stack([g2l, be2l])
    h_seq2 = _post_block(N_SEQ, (agg_inc2, agg_ct2), (dT_inc_d, dT_ct_d),
                         (W_inc2, W_ct2), (b_inc2, b_ct2), gb2s, None, 1000)
    h_lab2 = _post_block(N_LAB, (agg_bt2,), (dT_bt_d,), (W_bt2,), (b_bt2,),
                         gb2l, None, 1000)
    return (h_seq2, h_lab2)


# speedup vs baseline: 1.0523x; 1.0523x over previous
import functools

import jax
import jax.numpy as jnp
from jax import lax
from jax.experimental import pallas as pl
from jax.experimental.pallas import tpu as pltpu
from jax.experimental.pallas import tpu_sc as plsc

N_SEQ = 50000
N_LAB = 5000
D = 128
E_BT = 100000
E_INC = 100000
E_CT = 400000

NC = 2
NS = 16
L = 16
EW = 2000
DW = 2048

_f32 = jnp.float32
_i32 = jnp.int32

_DEG_JOBS = (
    (E_BT, N_SEQ, 50048),
    (E_BT, N_LAB, 5120),
    (E_INC, N_LAB, 5120),
    (E_INC, N_SEQ, 50048),
    (E_CT, N_SEQ, 50048),
    (E_CT, N_SEQ, 50048),
)


def _deg_body(bt_s, bt_d, inc_s, inc_d, ct_s, ct_d, o0, o1, o2, o3, o4, o5,
              idx2d, ones_v, zv, stg, *accs):
    core = lax.axis_index("c")
    sub = lax.axis_index("s")
    wid = sub * NC + core
    edge_refs = (bt_s, bt_d, inc_s, inc_d, ct_s, ct_d)
    out_refs = (o0, o1, o2, o3, o4, o5)

    def _fill(i, c):
        ones_v[pl.ds(i * L, L)] = jnp.ones((L,), _f32)
        zv[pl.ds(i * L, L)] = jnp.zeros((L,), _f32)
        return c
    lax.fori_loop(0, 128 // L, _fill, 0)

    for (E, N, NP), acc in zip(_DEG_JOBS, accs):
        share = NP // NS
        for j in range(share // 128):
            pltpu.sync_copy(zv, acc.at[pl.ds(sub * share + j * 128, 128)])
        rem = share % 128
        if rem:
            pltpu.sync_copy(zv.at[pl.ds(0, rem)],
                            acc.at[pl.ds(sub * share + share - rem, rem)])
    plsc.subcore_barrier()

    for eref, acc in zip(edge_refs, accs):
        nwin = eref.shape[0]
        nmine = (nwin - wid + 31) // 32

        def _win(i, c, eref=eref, acc=acc):
            w = wid + i * 32
            pltpu.sync_copy(eref.at[w], idx2d)
            for r in range(DW // 128):
                pltpu.sync_copy(ones_v, acc.at[idx2d.at[r]], add=True)
            return c
        lax.fori_loop(0, nmine, _win, 0)
    plsc.subcore_barrier()

    SB = 784
    for (E, N, NP), out, acc in zip(_DEG_JOBS, out_refs, accs):
        share = NP // NS
        for j in range(share // SB):
            pltpu.sync_copy(acc.at[pl.ds(sub * share + j * SB, SB)], stg)
            pltpu.sync_copy(stg,
                            out.at[pl.ds(core * NP + sub * share + j * SB,
                                         SB)])
        rem = share % SB
        if rem:
            off = share - rem
            pltpu.sync_copy(acc.at[pl.ds(sub * share + off, rem)],
                            stg.at[pl.ds(0, rem)])
            pltpu.sync_copy(stg.at[pl.ds(0, rem)],
                            out.at[pl.ds(core * NP + sub * share + off,
                                         rem)])


def _make_deg_kernel():
    out_type = tuple(jax.ShapeDtypeStruct((NC * np_,), _f32)
                     for (_, _, np_) in _DEG_JOBS)
    scratch = [
        pltpu.VMEM((DW // 128, 128), _i32),
        pltpu.VMEM((128,), _f32),
        pltpu.VMEM((128,), _f32),
        pltpu.VMEM((784,), _f32),
    ] + [pltpu.VMEM_SHARED((np_,), _f32) for (_, _, np_) in _DEG_JOBS]
    mesh = plsc.VectorSubcoreMesh(core_axis_name="c", subcore_axis_name="s",
                                  num_cores=NC, num_subcores=NS)
    return pl.kernel(_deg_body, out_type=out_type, mesh=mesh,
                     compiler_params=pltpu.CompilerParams(
                         needs_layout_passes=False),
                     scratch_types=scratch, name="sc_degrees")


def _segsum_stage(feat, src, dst, out, ew_s, ew_d, m_val, srcb, idx2d, rows,
                  zv, gsem, ssem, accum, core, sub, *, E, CHUNK, CPS):
    nwin = E // EW
    nmine = (nwin - sub + NS - 1) // NS
    AROWS = CHUNK + 128
    zshare = AROWS // NS
    fshare = CHUNK // NS
    B = 128
    ZR = zv.shape[0]

    for kk in range(CPS):
        chunk_id = core * CPS + kk
        lo = chunk_id * CHUNK

        for j in range(zshare // ZR):
            pltpu.sync_copy(zv, accum.at[pl.ds(sub * zshare + j * ZR, ZR)])
        rem = zshare % ZR
        if rem:
            pltpu.sync_copy(zv.at[pl.ds(0, rem)],
                            accum.at[pl.ds(sub * zshare + zshare - rem, rem)])
        plsc.subcore_barrier()

        def _win(i, cnt):
            w = sub + i * NS
            pltpu.sync_copy(src.at[pl.ds(w * EW, EW)], ew_s)
            pltpu.sync_copy(dst.at[pl.ds(w * EW, EW)], ew_d)

            def _grp(g, cnt):
                sv = ew_s[pl.ds(g * L, L)]
                dv = ew_d[pl.ds(g * L, L)]
                m = (dv >= lo) & (dv < lo + CHUNK)
                key = jnp.where(m, jnp.int32(0), jnp.int32(1))
                _, v2 = plsc.sort_key_val(key, sv * 16384 + (dv - lo))
                m_val[pl.ds(cnt, L)] = v2
                pc = plsc.all_reduce_population_count(m)
                return cnt + pc[0]
            return lax.fori_loop(0, EW // L, _grp, cnt)
        cnt = lax.fori_loop(0, nmine, _win, jnp.int32(0))

        lanes = lax.iota(_i32, L)
        for k in range(B // L):
            m_val[pl.ds(cnt + k * L, L)] = (
                (lanes + sub * L) * 16384 + (CHUNK + k * L + lanes))
        nb = (cnt + B - 1) // B

        def _build(bi, s):
            for k in range(B // L):
                v = m_val[pl.ds(bi * B + k * L, L)]
                srcb[s, pl.ds(k * L, L)] = lax.shift_right_logical(v, 14)
                idx2d[s, pl.ds(k * L, L)] = jnp.bitwise_and(v, 16383)

        @pl.when(nb >= 1)
        def _():
            _build(jnp.int32(0), jnp.int32(0))
            pltpu.async_copy(feat.at[srcb.at[0]], rows.at[pl.ds(0, B)],
                             gsem.at[0])

        def _batch(j, c):
            s = jnp.bitwise_and(j, 1)
            o = 1 - s
            rs = rows.at[pl.ds(s * B, B)]
            ro = rows.at[pl.ds(o * B, B)]

            @pl.when(j >= 1)
            def _():
                pltpu.make_async_copy(ro, accum.at[idx2d.at[o]], ssem).wait()

            @pl.when(j + 1 < nb)
            def _():
                _build(j + 1, o)
                pltpu.async_copy(feat.at[srcb.at[o]], ro, gsem.at[o])

            pltpu.make_async_copy(feat.at[srcb.at[s]], rs, gsem.at[s]).wait()
            pltpu.async_copy(rs, accum.at[idx2d.at[s]], ssem, add=True)
            return c
        lax.fori_loop(0, nb, _batch, 0)

        @pl.when(nb >= 1)
        def _():
            s = jnp.bitwise_and(nb - 1, 1)
            pltpu.make_async_copy(rows.at[pl.ds(s * B, B)],
                                  accum.at[idx2d.at[s]], ssem).wait()
        plsc.subcore_barrier()

        FB = rows.shape[0]
        for j in range(fshare // FB):
            pltpu.sync_copy(accum.at[pl.ds(sub * fshare + j * FB, FB)], rows)
            pltpu.sync_copy(rows, out.at[pl.ds(lo + sub * fshare + j * FB,
                                               FB)])
        frem = fshare % FB
        if frem:
            foff = fshare - frem
            pltpu.sync_copy(accum.at[pl.ds(sub * fshare + foff, frem)],
                            rows.at[pl.ds(0, frem)])
            pltpu.sync_copy(rows.at[pl.ds(0, frem)],
                            out.at[pl.ds(lo + sub * fshare + foff, frem)])
        plsc.subcore_barrier()


def _segsum_body(feat, src, dst, out, ew_s, ew_d, m_val, srcb, idx2d, rows,
                 zv, gsem, ssem, accum, *, E, CHUNK, CPS):
    core = lax.axis_index("c")
    sub = lax.axis_index("s")

    def _z(i, c):
        for k in range(D // L):
            zv[i, pl.ds(k * L, L)] = jnp.zeros((L,), _f32)
        return c
    lax.fori_loop(0, zv.shape[0], _z, 0)

    _segsum_stage(feat, src, dst, out, ew_s, ew_d, m_val, srcb, idx2d,
                  rows, zv, gsem, ssem, accum, core, sub,
                  E=E, CHUNK=CHUNK, CPS=CPS)


def _make_segsum_kernel(E, NDST, name):
    if NDST == N_LAB:
        nchunks, chunk = 2, 2560
    else:
        nchunks, chunk = 8, 6272
    ndst_pad = nchunks * chunk
    cps = nchunks // NC
    cap = ((E // EW + NS - 1) // NS) * EW + 128
    body = functools.partial(_segsum_body, E=E, CHUNK=chunk, CPS=cps)
    scratch = [
        pltpu.VMEM((EW,), _i32),
        pltpu.VMEM((EW,), _i32),
        pltpu.VMEM((cap,), _i32),
        pltpu.VMEM((2, 128), _i32),
        pltpu.VMEM((2, 128), _i32),
        pltpu.VMEM((256, D), _f32),
        pltpu.VMEM((32, D), _f32),
        pltpu.SemaphoreType.DMA((2,)),
        pltpu.SemaphoreType.DMA,
        pltpu.VMEM_SHARED((chunk + 128, D), _f32),
    ]
    mesh = plsc.VectorSubcoreMesh(core_axis_name="c", subcore_axis_name="s",
                                  num_cores=NC, num_subcores=NS)
    return pl.kernel(body,
                     out_type=jax.ShapeDtypeStruct((ndst_pad, D), _f32),
                     mesh=mesh,
                     compiler_params=pltpu.CompilerParams(
                         needs_layout_passes=False),
                     scratch_types=scratch, name=name)


def _inv_sqrt(dblk):
    return lax.rsqrt(jnp.maximum(jnp.sum(dblk, axis=1), 1.0))


def _prescale2_body(x_ref, da_ref, db_ref, oa_ref, ob_ref):
    x = x_ref[...]
    oa_ref[...] = x * _inv_sqrt(da_ref[...])[:, None]
    ob_ref[...] = x * _inv_sqrt(db_ref[...])[:, None]


def _prescale1_body(x_ref, da_ref, oa_ref):
    oa_ref[...] = x_ref[...] * _inv_sqrt(da_ref[...])[:, None]


def _prescale(x, degs, blk):
    n = x.shape[0]
    grid = n // blk
    xspec = pl.BlockSpec((blk, D), lambda i: (i, 0))
    dspec = pl.BlockSpec((blk, 2), lambda i: (i, 0))
    if len(degs) == 2:
        return pl.pallas_call(
            _prescale2_body, grid=(grid,),
            in_specs=[xspec, dspec, dspec],
            out_specs=(xspec, xspec),
            out_shape=(jax.ShapeDtypeStruct((n, D), _f32),) * 2,
        )(x, degs[0], degs[1])
    return pl.pallas_call(
        _prescale1_body, grid=(grid,),
        in_specs=[xspec, dspec],
        out_specs=xspec,
        out_shape=jax.ShapeDtypeStruct((n, D), _f32))(x, degs[0])


def _relu_block(a, b, da, db, wa, wb, bias):
    sa = _inv_sqrt(da)[:, None]
    sb = _inv_sqrt(db)[:, None]
    y = (jnp.dot(a * sa, wa, preferred_element_type=_f32)
         + jnp.dot(b * sb, wb, preferred_element_type=_f32)
         + bias[0, :][None, :] + bias[1, :][None, :]) * 0.5
    return jnp.maximum(y, 0.0)


def _post_stats_body(a_ref, b_ref, da_ref, db_ref, wa_ref, wb_ref, bias_ref,
                     stat_ref, acc_ref, *, grid):
    i = pl.program_id(0)
    y = _relu_block(a_ref[...], b_ref[...], da_ref[...], db_ref[...],
                    wa_ref[...], wb_ref[...], bias_ref[...])

    @pl.when(i == 0)
    def _():
        acc_ref[...] = jnp.zeros_like(acc_ref)

    s1 = jnp.sum(y, axis=0)
    s2 = jnp.sum(y * y, axis=0)
    acc_ref[...] += jnp.concatenate([s1[None, :], s2[None, :]], axis=0)

    @pl.when(i == grid - 1)
    def _():
        stat_ref[...] = acc_ref[...]


def _post_apply2_body(a_ref, b_ref, da_ref, db_ref, wa_ref, wb_ref, bias_ref,
                      stat_ref, gb_ref, so1_ref, so2_ref, o1_ref, o2_ref, *,
                      n):
    y = _relu_block(a_ref[...], b_ref[...], da_ref[...], db_ref[...],
                    wa_ref[...], wb_ref[...], bias_ref[...])
    mu = stat_ref[0, :] / n
    var = stat_ref[1, :] / n - mu * mu
    h = (y - mu[None, :]) * lax.rsqrt(var + 1e-5)[None, :]
    h = h * gb_ref[0, :][None, :] + gb_ref[1, :][None, :]
    o1_ref[...] = h * _inv_sqrt(so1_ref[...])[:, None]
    o2_ref[...] = h * _inv_sqrt(so2_ref[...])[:, None]


def _post_apply1_body(a_ref, b_ref, da_ref, db_ref, wa_ref, wb_ref, bias_ref,
                      stat_ref, gb_ref, o1_ref, *, n):
    y = _relu_block(a_ref[...], b_ref[...], da_ref[...], db_ref[...],
                    wa_ref[...], wb_ref[...], bias_ref[...])
    mu = stat_ref[0, :] / n
    var = stat_ref[1, :] / n - mu * mu
    h = (y - mu[None, :]) * lax.rsqrt(var + 1e-5)[None, :]
    o1_ref[...] = h * gb_ref[0, :][None, :] + gb_ref[1, :][None, :]


def _post_block(n, aggs, degs_in, Ws, biases, gamma_beta, out_scale, blk):
    grid = n // blk
    if len(aggs) == 1:
        aggs = (aggs[0], aggs[0])
        degs_in = (degs_in[0], degs_in[0])
        Ws = (Ws[0], Ws[0])
        biases = (biases[0], biases[0])
    bias = jnp.concatenate([biases[0][None, :], biases[1][None, :]], axis=0)
    aspec = pl.BlockSpec((blk, D), lambda i: (i, 0))
    dspec = pl.BlockSpec((blk, 2), lambda i: (i, 0))
    wspec = pl.BlockSpec((D, D), lambda i: (0, 0))
    cspec = pl.BlockSpec((2, D), lambda i: (0, 0))
    args = (aggs[0], aggs[1], degs_in[0], degs_in[1], Ws[0], Ws[1], bias)
    stats = pl.pallas_call(
        functools.partial(_post_stats_body, grid=grid),
        grid=(grid,),
        in_specs=[aspec, aspec, dspec, dspec, wspec, wspec, cspec],
        out_specs=cspec,
        out_shape=jax.ShapeDtypeStruct((2, D), _f32),
        scratch_shapes=[pltpu.VMEM((2, D), _f32)],
    )(*args)
    if out_scale is not None:
        return pl.pallas_call(
            functools.partial(_post_apply2_body, n=float(n)),
            grid=(grid,),
            in_specs=[aspec, aspec, dspec, dspec, wspec, wspec, cspec, cspec,
                      cspec, dspec, dspec],
            out_specs=(aspec, aspec),
            out_shape=(jax.ShapeDtypeStruct((n, D), _f32),) * 2,
        )(*args, stats, gamma_beta, out_scale[0], out_scale[1])
    return pl.pallas_call(
        functools.partial(_post_apply1_body, n=float(n)),
        grid=(grid,),
        in_specs=[aspec, aspec, dspec, dspec, wspec, wspec, cspec, cspec,
                  cspec],
        out_specs=aspec,
        out_shape=jax.ShapeDtypeStruct((n, D), _f32),
    )(*args, stats, gamma_beta)


_deg_kernel = _make_deg_kernel()
_seg_bt = _make_segsum_kernel(E_BT, N_LAB, "sc_segsum_bt")
_seg_inc = _make_segsum_kernel(E_INC, N_SEQ, "sc_segsum_inc")
_seg_ct = _make_segsum_kernel(E_CT, N_SEQ, "sc_segsum_ct")


def _pad_edges(idx, n_fill):
    e = idx.shape[0]
    ep = (e + DW - 1) // DW * DW
    out = jnp.concatenate([idx, jnp.full((ep - e,), n_fill, _i32)])
    return out.reshape(ep // DW, DW // 128, 128)


def kernel(x_sequence, x_label, bt_src, bt_dst, inc_src, inc_dst, ct_src,
           ct_dst, W_bt1, b_bt1, W_inc1, b_inc1, W_ct1, b_ct1, W_bt2, b_bt2,
           W_inc2, b_inc2, W_ct2, b_ct2, g1s, be1s, g1l, be1l, g2s, be2s,
           g2l, be2l):
    degs = _deg_kernel(_pad_edges(bt_src, N_SEQ), _pad_edges(bt_dst, N_LAB),
                       _pad_edges(inc_src, N_LAB), _pad_edges(inc_dst, N_SEQ),
                       _pad_edges(ct_src, N_SEQ), _pad_edges(ct_dst, N_SEQ))
    dT_bt_s, dT_bt_d, dT_inc_s, dT_inc_d, dT_ct_s, dT_ct_d = (
        d.reshape(NC, -1).T for d in degs)

    feat_bt1, feat_ct1 = _prescale(x_sequence, (dT_bt_s, dT_ct_s), 1000)
    feat_inc1 = _prescale(x_label, (dT_inc_s,), 1000)

    agg_bt1 = _seg_bt(feat_bt1, bt_src, bt_dst)
    agg_inc1 = _seg_inc(feat_inc1, inc_src, inc_dst)
    agg_ct1 = _seg_ct(feat_ct1, ct_src, ct_dst)

    gb1s = jnp.stack([g1s, be1s])
    gb1l = jnp.stack([g1l, be1l])
    feat_bt2, feat_ct2 = _post_block(
        N_SEQ, (agg_inc1, agg_ct1), (dT_inc_d, dT_ct_d), (W_inc1, W_ct1),
        (b_inc1, b_ct1), gb1s, (dT_bt_s, dT_ct_s), 1000)
    feat_inc2 = _post_block(N_LAB, (agg_bt1,), (dT_bt_d,), (W_bt1,),
                            (b_bt1,), gb1l, (dT_inc_s, dT_inc_s), 1000)[0]

    agg_bt2 = _seg_bt(feat_bt2, bt_src, bt_dst)
    agg_inc2 = _seg_inc(feat_inc2, inc_src, inc_dst)
    agg_ct2 = _seg_ct(feat_ct2, ct_src, ct_dst)

    gb2s = jnp.stack([g2s, be2s])
    gb2l = jnp.stack([g2l, be2l])
    h_seq2 = _post_block(N_SEQ, (agg_inc2, agg_ct2), (dT_inc_d, dT_ct_d),
                         (W_inc2, W_ct2), (b_inc2, b_ct2), gb2s, None, 1000)
    h_lab2 = _post_block(N_LAB, (agg_bt2,), (dT_bt_d,), (W_bt2,), (b_bt2,),
                         gb2l, None, 1000)
    return (h_seq2, h_lab2)


# speedup vs baseline: 1.0675x; 1.0144x over previous
import functools

import jax
import jax.numpy as jnp
from jax import lax
from jax.experimental import pallas as pl
from jax.experimental.pallas import tpu as pltpu
from jax.experimental.pallas import tpu_sc as plsc

N_SEQ = 50000
N_LAB = 5000
D = 128
E_BT = 100000
E_INC = 100000
E_CT = 400000

NC = 2
NS = 16
L = 16
EW = 2000
DW = 2048

_f32 = jnp.float32
_i32 = jnp.int32

_DEG_JOBS = (
    (E_BT, N_SEQ, 50048),
    (E_BT, N_LAB, 5120),
    (E_INC, N_LAB, 5120),
    (E_INC, N_SEQ, 50048),
    (E_CT, N_SEQ, 50048),
    (E_CT, N_SEQ, 50048),
)


def _deg_body(bt_s, bt_d, inc_s, inc_d, ct_s, ct_d, o0, o1, o2, o3, o4, o5,
              idx2d, ones_v, zv, stg, dsem, *accs):
    core = lax.axis_index("c")
    sub = lax.axis_index("s")
    wid = sub * NC + core
    edge_refs = (bt_s, bt_d, inc_s, inc_d, ct_s, ct_d)
    out_refs = (o0, o1, o2, o3, o4, o5)

    def _fill(i, c):
        ones_v[pl.ds(i * L, L)] = jnp.ones((L,), _f32)
        zv[pl.ds(i * L, L)] = jnp.zeros((L,), _f32)
        return c
    lax.fori_loop(0, 128 // L, _fill, 0)

    for (E, N, NP), acc in zip(_DEG_JOBS, accs):
        share = NP // NS
        for j in range(share // 128):
            pltpu.sync_copy(zv, acc.at[pl.ds(sub * share + j * 128, 128)])
        rem = share % 128
        if rem:
            pltpu.sync_copy(zv.at[pl.ds(0, rem)],
                            acc.at[pl.ds(sub * share + share - rem, rem)])
    plsc.subcore_barrier()

    for eref, acc in zip(edge_refs, accs):
        nwin = eref.shape[0]
        nmine = (nwin - wid + 31) // 32

        def _win(i, c, eref=eref, acc=acc):
            w = wid + i * 32
            pltpu.sync_copy(eref.at[w], idx2d)
            for r in range(DW // 128):
                pltpu.async_copy(ones_v, acc.at[idx2d.at[r]], dsem, add=True)
            for r in range(DW // 128):
                pltpu.make_async_copy(ones_v, acc.at[idx2d.at[0]],
                                      dsem).wait()
            return c
        lax.fori_loop(0, nmine, _win, 0)
    plsc.subcore_barrier()

    SB = 784
    for (E, N, NP), out, acc in zip(_DEG_JOBS, out_refs, accs):
        share = NP // NS
        for j in range(share // SB):
            pltpu.sync_copy(acc.at[pl.ds(sub * share + j * SB, SB)], stg)
            pltpu.sync_copy(stg,
                            out.at[pl.ds(core * NP + sub * share + j * SB,
                                         SB)])
        rem = share % SB
        if rem:
            off = share - rem
            pltpu.sync_copy(acc.at[pl.ds(sub * share + off, rem)],
                            stg.at[pl.ds(0, rem)])
            pltpu.sync_copy(stg.at[pl.ds(0, rem)],
                            out.at[pl.ds(core * NP + sub * share + off,
                                         rem)])


def _make_deg_kernel():
    out_type = tuple(jax.ShapeDtypeStruct((NC * np_,), _f32)
                     for (_, _, np_) in _DEG_JOBS)
    scratch = [
        pltpu.VMEM((DW // 128, 128), _i32),
        pltpu.VMEM((128,), _f32),
        pltpu.VMEM((128,), _f32),
        pltpu.VMEM((784,), _f32),
        pltpu.SemaphoreType.DMA,
    ] + [pltpu.VMEM_SHARED((np_,), _f32) for (_, _, np_) in _DEG_JOBS]
    mesh = plsc.VectorSubcoreMesh(core_axis_name="c", subcore_axis_name="s",
                                  num_cores=NC, num_subcores=NS)
    return pl.kernel(_deg_body, out_type=out_type, mesh=mesh,
                     compiler_params=pltpu.CompilerParams(
                         needs_layout_passes=False),
                     scratch_types=scratch, name="sc_degrees")


def _segsum_stage(feat, src, dst, out, ew_s, ew_d, m_val, srcb, idx2d,
                  rows, zv, gsem, ssem, accum, core, sub, *, E, CHUNK, CPS):
    nwin = E // EW
    nmine = (nwin - sub + NS - 1) // NS
    AROWS = CHUNK + 128
    zshare = AROWS // NS
    fshare = CHUNK // NS
    B = 128
    ZR = zv.shape[0]

    for kk in range(CPS):
        chunk_id = core * CPS + kk
        lo = chunk_id * CHUNK

        for j in range(zshare // ZR):
            pltpu.sync_copy(zv, accum.at[pl.ds(sub * zshare + j * ZR, ZR)])
        rem = zshare % ZR
        if rem:
            pltpu.sync_copy(zv.at[pl.ds(0, rem)],
                            accum.at[pl.ds(sub * zshare + zshare - rem, rem)])
        plsc.subcore_barrier()

        def _win(i, cnt):
            w = sub + i * NS
            pltpu.sync_copy(src.at[pl.ds(w * EW, EW)], ew_s)
            pltpu.sync_copy(dst.at[pl.ds(w * EW, EW)], ew_d)

            def _grp(g, cnt):
                sv = ew_s[pl.ds(g * L, L)]
                dv = ew_d[pl.ds(g * L, L)]
                m = (dv >= lo) & (dv < lo + CHUNK)
                key = jnp.where(m, jnp.int32(0), jnp.int32(1))
                _, v2 = plsc.sort_key_val(key, sv * 16384 + (dv - lo))
                m_val[pl.ds(cnt, L)] = v2
                pc = plsc.all_reduce_population_count(m)
                return cnt + pc[0]
            return lax.fori_loop(0, EW // L, _grp, cnt)
        cnt = lax.fori_loop(0, nmine, _win, jnp.int32(0))

        lanes = lax.iota(_i32, L)
        for k in range(B // L):
            m_val[pl.ds(cnt + k * L, L)] = (
                (lanes + sub * L) * 16384 + (CHUNK + k * L + lanes))
        nb = (cnt + B - 1) // B

        def _build(bi, s):
            for k in range(B // L):
                v = m_val[pl.ds(bi * B + k * L, L)]
                srcb[s, pl.ds(k * L, L)] = lax.shift_right_logical(v, 14)
                idx2d[s, pl.ds(k * L, L)] = jnp.bitwise_and(v, 16383)

        @pl.when(nb >= 1)
        def _():
            _build(jnp.int32(0), jnp.int32(0))
            pltpu.async_copy(feat.at[srcb.at[0]], rows.at[pl.ds(0, B)],
                             gsem.at[0])

        def _batch(j, c):
            s = jnp.bitwise_and(j, 1)
            o = 1 - s
            rs = rows.at[pl.ds(s * B, B)]
            ro = rows.at[pl.ds(o * B, B)]

            @pl.when(j >= 1)
            def _():
                pltpu.make_async_copy(ro, accum.at[idx2d.at[o]], ssem).wait()

            @pl.when(j + 1 < nb)
            def _():
                _build(j + 1, o)
                pltpu.async_copy(feat.at[srcb.at[o]], ro, gsem.at[o])

            pltpu.make_async_copy(feat.at[srcb.at[s]], rs, gsem.at[s]).wait()
            pltpu.async_copy(rs, accum.at[idx2d.at[s]], ssem, add=True)
            return c
        lax.fori_loop(0, nb, _batch, 0)

        @pl.when(nb >= 1)
        def _():
            s = jnp.bitwise_and(nb - 1, 1)
            pltpu.make_async_copy(rows.at[pl.ds(s * B, B)],
                                  accum.at[idx2d.at[s]], ssem).wait()
        plsc.subcore_barrier()

        FB = rows.shape[0]
        for j in range(fshare // FB):
            pltpu.sync_copy(accum.at[pl.ds(sub * fshare + j * FB, FB)], rows)
            pltpu.sync_copy(rows, out.at[pl.ds(lo + sub * fshare + j * FB,
                                               FB)])
        frem = fshare % FB
        if frem:
            foff = fshare - frem
            pltpu.sync_copy(accum.at[pl.ds(sub * fshare + foff, frem)],
                            rows.at[pl.ds(0, frem)])
            pltpu.sync_copy(rows.at[pl.ds(0, frem)],
                            out.at[pl.ds(lo + sub * fshare + foff, frem)])
        plsc.subcore_barrier()


def _segsum_body(feat, src, dst, out, ew_s, ew_d, m_val, srcb, idx2d,
                 rows, zv, gsem, ssem, accum, *, E, CHUNK, CPS):
    core = lax.axis_index("c")
    sub = lax.axis_index("s")

    def _z(i, c):
        for k in range(D // L):
            zv[i, pl.ds(k * L, L)] = jnp.zeros((L,), _f32)
        return c
    lax.fori_loop(0, zv.shape[0], _z, 0)

    _segsum_stage(feat, src, dst, out, ew_s, ew_d, m_val, srcb, idx2d,
                  rows, zv, gsem, ssem, accum, core, sub,
                  E=E, CHUNK=CHUNK, CPS=CPS)


def _make_segsum_kernel(E, NDST, name):
    if NDST == N_LAB:
        nchunks, chunk = 2, 2560
    else:
        nchunks, chunk = 8, 6272
    ndst_pad = nchunks * chunk
    cps = nchunks // NC
    cap = ((E // EW + NS - 1) // NS) * EW + 128
    body = functools.partial(_segsum_body, E=E, CHUNK=chunk, CPS=cps)
    scratch = [
        pltpu.VMEM((EW,), _i32),
        pltpu.VMEM((EW,), _i32),
        pltpu.VMEM((cap,), _i32),
        pltpu.VMEM((2, 128), _i32),
        pltpu.VMEM((2, 128), _i32),
        pltpu.VMEM((256, D), _f32),
        pltpu.VMEM((32, D), _f32),
        pltpu.SemaphoreType.DMA((2,)),
        pltpu.SemaphoreType.DMA,
        pltpu.VMEM_SHARED((chunk + 128, D), _f32),
    ]
    mesh = plsc.VectorSubcoreMesh(core_axis_name="c", subcore_axis_name="s",
                                  num_cores=NC, num_subcores=NS)
    return pl.kernel(body,
                     out_type=jax.ShapeDtypeStruct((ndst_pad, D), _f32),
                     mesh=mesh,
                     compiler_params=pltpu.CompilerParams(
                         needs_layout_passes=False),
                     scratch_types=scratch, name=name)


def _inv_sqrt(dblk):
    return lax.rsqrt(jnp.maximum(jnp.sum(dblk, axis=1), 1.0))


def _prescale2_body(x_ref, da_ref, db_ref, oa_ref, ob_ref):
    x = x_ref[...]
    oa_ref[...] = x * _inv_sqrt(da_ref[...])[:, None]
    ob_ref[...] = x * _inv_sqrt(db_ref[...])[:, None]


def _prescale1_body(x_ref, da_ref, oa_ref):
    oa_ref[...] = x_ref[...] * _inv_sqrt(da_ref[...])[:, None]


def _prescale(x, degs, blk):
    n = x.shape[0]
    grid = n // blk
    xspec = pl.BlockSpec((blk, D), lambda i: (i, 0))
    dspec = pl.BlockSpec((blk, 2), lambda i: (i, 0))
    if len(degs) == 2:
        return pl.pallas_call(
            _prescale2_body, grid=(grid,),
            in_specs=[xspec, dspec, dspec],
            out_specs=(xspec, xspec),
            out_shape=(jax.ShapeDtypeStruct((n, D), _f32),) * 2,
        )(x, degs[0], degs[1])
    return pl.pallas_call(
        _prescale1_body, grid=(grid,),
        in_specs=[xspec, dspec],
        out_specs=xspec,
        out_shape=jax.ShapeDtypeStruct((n, D), _f32))(x, degs[0])


def _relu_block(a, b, da, db, wa, wb, bias):
    sa = _inv_sqrt(da)[:, None]
    sb = _inv_sqrt(db)[:, None]
    y = (jnp.dot(a * sa, wa, preferred_element_type=_f32)
         + jnp.dot(b * sb, wb, preferred_element_type=_f32)
         + bias[0, :][None, :] + bias[1, :][None, :]) * 0.5
    return jnp.maximum(y, 0.0)


def _post_stats_body(a_ref, b_ref, da_ref, db_ref, wa_ref, wb_ref, bias_ref,
                     stat_ref, acc_ref, *, grid):
    i = pl.program_id(0)
    y = _relu_block(a_ref[...], b_ref[...], da_ref[...], db_ref[...],
                    wa_ref[...], wb_ref[...], bias_ref[...])

    @pl.when(i == 0)
    def _():
        acc_ref[...] = jnp.zeros_like(acc_ref)

    s1 = jnp.sum(y, axis=0)
    s2 = jnp.sum(y * y, axis=0)
    acc_ref[...] += jnp.concatenate([s1[None, :], s2[None, :]], axis=0)

    @pl.when(i == grid - 1)
    def _():
        stat_ref[...] = acc_ref[...]


def _post_apply2_body(a_ref, b_ref, da_ref, db_ref, wa_ref, wb_ref, bias_ref,
                      stat_ref, gb_ref, so1_ref, so2_ref, o1_ref, o2_ref, *,
                      n):
    y = _relu_block(a_ref[...], b_ref[...], da_ref[...], db_ref[...],
                    wa_ref[...], wb_ref[...], bias_ref[...])
    mu = stat_ref[0, :] / n
    var = stat_ref[1, :] / n - mu * mu
    h = (y - mu[None, :]) * lax.rsqrt(var + 1e-5)[None, :]
    h = h * gb_ref[0, :][None, :] + gb_ref[1, :][None, :]
    o1_ref[...] = h * _inv_sqrt(so1_ref[...])[:, None]
    o2_ref[...] = h * _inv_sqrt(so2_ref[...])[:, None]


def _post_apply1_body(a_ref, b_ref, da_ref, db_ref, wa_ref, wb_ref, bias_ref,
                      stat_ref, gb_ref, o1_ref, *, n):
    y = _relu_block(a_ref[...], b_ref[...], da_ref[...], db_ref[...],
                    wa_ref[...], wb_ref[...], bias_ref[...])
    mu = stat_ref[0, :] / n
    var = stat_ref[1, :] / n - mu * mu
    h = (y - mu[None, :]) * lax.rsqrt(var + 1e-5)[None, :]
    o1_ref[...] = h * gb_ref[0, :][None, :] + gb_ref[1, :][None, :]


def _post_block(n, aggs, degs_in, Ws, biases, gamma_beta, out_scale, blk):
    grid = n // blk
    if len(aggs) == 1:
        aggs = (aggs[0], aggs[0])
        degs_in = (degs_in[0], degs_in[0])
        Ws = (Ws[0], Ws[0])
        biases = (biases[0], biases[0])
    bias = jnp.concatenate([biases[0][None, :], biases[1][None, :]], axis=0)
    aspec = pl.BlockSpec((blk, D), lambda i: (i, 0))
    dspec = pl.BlockSpec((blk, 2), lambda i: (i, 0))
    wspec = pl.BlockSpec((D, D), lambda i: (0, 0))
    cspec = pl.BlockSpec((2, D), lambda i: (0, 0))
    args = (aggs[0], aggs[1], degs_in[0], degs_in[1], Ws[0], Ws[1], bias)
    stats = pl.pallas_call(
        functools.partial(_post_stats_body, grid=grid),
        grid=(grid,),
        in_specs=[aspec, aspec, dspec, dspec, wspec, wspec, cspec],
        out_specs=cspec,
        out_shape=jax.ShapeDtypeStruct((2, D), _f32),
        scratch_shapes=[pltpu.VMEM((2, D), _f32)],
    )(*args)
    if out_scale is not None:
        return pl.pallas_call(
            functools.partial(_post_apply2_body, n=float(n)),
            grid=(grid,),
            in_specs=[aspec, aspec, dspec, dspec, wspec, wspec, cspec, cspec,
                      cspec, dspec, dspec],
            out_specs=(aspec, aspec),
            out_shape=(jax.ShapeDtypeStruct((n, D), _f32),) * 2,
        )(*args, stats, gamma_beta, out_scale[0], out_scale[1])
    return pl.pallas_call(
        functools.partial(_post_apply1_body, n=float(n)),
        grid=(grid,),
        in_specs=[aspec, aspec, dspec, dspec, wspec, wspec, cspec, cspec,
                  cspec],
        out_specs=aspec,
        out_shape=jax.ShapeDtypeStruct((n, D), _f32),
    )(*args, stats, gamma_beta)


_deg_kernel = _make_deg_kernel()
_seg_bt = _make_segsum_kernel(E_BT, N_LAB, "sc_segsum_bt")
_seg_inc = _make_segsum_kernel(E_INC, N_SEQ, "sc_segsum_inc")
_seg_ct = _make_segsum_kernel(E_CT, N_SEQ, "sc_segsum_ct")


def _pad_edges(idx, n_fill):
    e = idx.shape[0]
    ep = (e + DW - 1) // DW * DW
    out = jnp.concatenate([idx, jnp.full((ep - e,), n_fill, _i32)])
    return out.reshape(ep // DW, DW // 128, 128)


def kernel(x_sequence, x_label, bt_src, bt_dst, inc_src, inc_dst, ct_src,
           ct_dst, W_bt1, b_bt1, W_inc1, b_inc1, W_ct1, b_ct1, W_bt2, b_bt2,
           W_inc2, b_inc2, W_ct2, b_ct2, g1s, be1s, g1l, be1l, g2s, be2s,
           g2l, be2l):
    degs = _deg_kernel(_pad_edges(bt_src, N_SEQ), _pad_edges(bt_dst, N_LAB),
                       _pad_edges(inc_src, N_LAB), _pad_edges(inc_dst, N_SEQ),
                       _pad_edges(ct_src, N_SEQ), _pad_edges(ct_dst, N_SEQ))
    dT_bt_s, dT_bt_d, dT_inc_s, dT_inc_d, dT_ct_s, dT_ct_d = (
        d.reshape(NC, -1).T for d in degs)

    feat_bt1, feat_ct1 = _prescale(x_sequence, (dT_bt_s, dT_ct_s), 1000)
    feat_inc1 = _prescale(x_label, (dT_inc_s,), 1000)

    agg_bt1 = _seg_bt(feat_bt1, bt_src, bt_dst)
    agg_inc1 = _seg_inc(feat_inc1, inc_src, inc_dst)
    agg_ct1 = _seg_ct(feat_ct1, ct_src, ct_dst)

    gb1s = jnp.stack([g1s, be1s])
    gb1l = jnp.stack([g1l, be1l])
    feat_bt2, feat_ct2 = _post_block(
        N_SEQ, (agg_inc1, agg_ct1), (dT_inc_d, dT_ct_d), (W_inc1, W_ct1),
        (b_inc1, b_ct1), gb1s, (dT_bt_s, dT_ct_s), 1000)
    feat_inc2 = _post_block(N_LAB, (agg_bt1,), (dT_bt_d,), (W_bt1,),
                            (b_bt1,), gb1l, (dT_inc_s, dT_inc_s), 1000)[0]

    agg_bt2 = _seg_bt(feat_bt2, bt_src, bt_dst)
    agg_inc2 = _seg_inc(feat_inc2, inc_src, inc_dst)
    agg_ct2 = _seg_ct(feat_ct2, ct_src, ct_dst)

    gb2s = jnp.stack([g2s, be2s])
    gb2l = jnp.stack([g2l, be2l])
    h_seq2 = _post_block(N_SEQ, (agg_inc2, agg_ct2), (dT_inc_d, dT_ct_d),
                         (W_inc2, W_ct2), (b_inc2, b_ct2), gb2s, None, 1000)
    h_lab2 = _post_block(N_LAB, (agg_bt2,), (dT_bt_d,), (W_bt2,), (b_bt2,),
                         gb2l, None, 1000)
    return (h_seq2, h_lab2)
